# Initial kernel scaffold; baseline (speedup 1.0000x reference)
import jax
import jax.numpy as jnp
from jax.experimental import pallas as pl


def kernel(x, edge_index, batch, tables, W1, b1, W2, b2, W3, b3, lin_w, lin_b):
    raise NotImplementedError("write your pallas kernel here")


# speedup vs baseline: 6.1004x; 6.1004x over previous
import functools

import jax
import jax.numpy as jnp
from jax import lax
from jax.experimental import pallas as pl
from jax.experimental.pallas import tpu as pltpu
from jax.experimental.pallas import tpu_sc as plsc

N = 10000
E = 320000
H = 128
NF = 9
V = 64
G = 256

NC = 2
NS = 16
NW = NC * NS

CHUNK = 128
CH_PER_W = 80
EPW = CHUNK * CH_PER_W
E_PAD = NW * EPW
NP = 10240
RPT = NP // NS
IC = 128
GPT = G // NS
PC = 80
PNCH = N // PC

_mesh = plsc.VectorSubcoreMesh(core_axis_name="c", subcore_axis_name="s")


def _zero_rows(buf, nrows, ncol16):
    z16 = jnp.zeros((16,), jnp.float32)
    def body(r, _):
        for k in range(ncol16):
            buf[r, pl.ds(k * 16, 16)] = z16
        return 0
    lax.fori_loop(0, nrows, body, 0)


def _fill_ones(buf, nrows, ncol16):
    o16 = jnp.ones((16,), jnp.float32)
    def body(r, _):
        for k in range(ncol16):
            buf[r, pl.ds(k * 16, 16)] = o16
        return 0
    lax.fori_loop(0, nrows, body, 0)


@functools.partial(
    pl.kernel,
    out_type=jax.ShapeDtypeStruct((NC, NP, H), jnp.float32),
    mesh=_mesh,
    scratch_types=[
        pltpu.VMEM((CHUNK,), jnp.int32),
        pltpu.VMEM((CHUNK, H), jnp.float32),
        pltpu.VMEM_SHARED((NP, H), jnp.float32),
    ],
)
def _deg_pass(dst_hbm, deg_hbm, didx, obuf, deg2d):
    c = lax.axis_index("c")
    s = lax.axis_index("s")
    wid = c * NS + s

    _zero_rows(obuf, IC, H // 16)
    for j in range(RPT // IC):
        pltpu.sync_copy(obuf.at[pl.ds(0, IC)],
                        deg2d.at[pl.ds(s * RPT + j * IC, IC)])
    _fill_ones(obuf, CHUNK, H // 16)
    plsc.subcore_barrier()

    base0 = wid * EPW

    def chunk(i, _):
        eb = pl.multiple_of(base0 + i * CHUNK, CHUNK)
        pltpu.sync_copy(dst_hbm.at[pl.ds(eb, CHUNK)], didx)
        pltpu.sync_copy(obuf, deg2d.at[didx], add=True)
        return 0

    lax.fori_loop(0, CH_PER_W, chunk, 0)
    plsc.subcore_barrier()

    rb = s * RPT
    pltpu.sync_copy(deg2d.at[pl.ds(rb, RPT)], deg_hbm.at[c, pl.ds(rb, RPT)])


@functools.partial(
    pl.kernel,
    out_type=jax.ShapeDtypeStruct((NC, NP, H), jnp.float32),
    mesh=_mesh,
    scratch_types=[
        pltpu.VMEM((CHUNK,), jnp.int32),
        pltpu.VMEM((CHUNK,), jnp.int32),
        pltpu.VMEM((CHUNK, H), jnp.float32),
        pltpu.VMEM_SHARED((NP, H), jnp.float32),
        pltpu.SemaphoreType.DMA,
    ],
)
def _conv_pass(src_hbm, dst_hbm, hs_hbm, out_hbm, sidx, didx, rows, acc, sem):
    c = lax.axis_index("c")
    s = lax.axis_index("s")
    wid = c * NS + s

    _zero_rows(rows, CHUNK, H // 16)
    for j in range(RPT // IC):
        pltpu.sync_copy(rows.at[pl.ds(0, IC)],
                        acc.at[pl.ds(s * RPT + j * IC, IC)])
    plsc.subcore_barrier()

    base0 = wid * EPW

    def chunk(i, _):
        eb = pl.multiple_of(base0 + i * CHUNK, CHUNK)
        pltpu.sync_copy(src_hbm.at[pl.ds(eb, CHUNK)], sidx)
        pltpu.sync_copy(dst_hbm.at[pl.ds(eb, CHUNK)], didx)
        pltpu.async_copy(hs_hbm.at[sidx], rows, sem).wait()
        pltpu.sync_copy(rows, acc.at[didx], add=True)
        return 0

    lax.fori_loop(0, CH_PER_W, chunk, 0)
    plsc.subcore_barrier()

    rb = s * RPT
    pltpu.sync_copy(acc.at[pl.ds(rb, RPT)], out_hbm.at[c, pl.ds(rb, RPT)])


@functools.partial(
    pl.kernel,
    out_type=(jax.ShapeDtypeStruct((G, H), jnp.float32),
              jax.ShapeDtypeStruct((G, H), jnp.float32)),
    mesh=_mesh,
    scratch_types=[
        pltpu.VMEM((PC,), jnp.int32),
        pltpu.VMEM((PC, H), jnp.float32),
        pltpu.VMEM((PC, H), jnp.float32),
        pltpu.VMEM_SHARED((G, H), jnp.float32),
        pltpu.VMEM_SHARED((G, H), jnp.float32),
    ],
)
def _pool_pass(h_hbm, batch_hbm, pooled_hbm, counts_hbm,
               bidx, prow, cbuf, pooled2d, counts2d):
    c = lax.axis_index("c")
    s = lax.axis_index("s")
    nch = (PNCH - s + NS - 1) // NS

    @pl.when(c == 0)
    def _():
        _zero_rows(prow, GPT, H // 16)
        pltpu.sync_copy(prow.at[pl.ds(0, GPT)],
                        pooled2d.at[pl.ds(s * GPT, GPT)])
        plsc.subcore_barrier()

        def chunk(j, _):
            eb = pl.multiple_of((s + j * NS) * PC, PC)
            pltpu.sync_copy(batch_hbm.at[pl.ds(eb, PC)], bidx)
            pltpu.sync_copy(h_hbm.at[pl.ds(eb, PC)], prow)
            pltpu.sync_copy(prow, pooled2d.at[bidx], add=True)
            return 0

        lax.fori_loop(0, nch, chunk, 0)
        plsc.subcore_barrier()
        pltpu.sync_copy(pooled2d.at[pl.ds(s * GPT, GPT)],
                        pooled_hbm.at[pl.ds(s * GPT, GPT)])

    @pl.when(c == 1)
    def _():
        _zero_rows(cbuf, GPT, H // 16)
        pltpu.sync_copy(cbuf.at[pl.ds(0, GPT)],
                        counts2d.at[pl.ds(s * GPT, GPT)])
        _fill_ones(cbuf, PC, H // 16)
        plsc.subcore_barrier()

        def chunk(j, _):
            eb = pl.multiple_of((s + j * NS) * PC, PC)
            pltpu.sync_copy(batch_hbm.at[pl.ds(eb, PC)], bidx)
            pltpu.sync_copy(cbuf, counts2d.at[bidx], add=True)
            return 0

        lax.fori_loop(0, nch, chunk, 0)
        plsc.subcore_barrier()
        pltpu.sync_copy(counts2d.at[pl.ds(s * GPT, GPT)],
                        counts_hbm.at[pl.ds(s * GPT, GPT)])


BLK = 2000


def _dinv_blk(d0_ref, d1_ref):
    deg = d0_ref[0, :, 0:1] + d1_ref[0, :, 0:1] + 1.0
    return lax.rsqrt(deg)


def _tc0_body(x_ref, tab_ref, d0_ref, d1_ref, w_ref, out_ref):
    h = jnp.zeros((BLK, H), jnp.float32)
    for f in range(NF):
        xf = x_ref[:, f:f + 1]
        oh = (xf == lax.broadcasted_iota(jnp.int32, (BLK, V), 1))
        h = h + jnp.dot(oh.astype(jnp.float32), tab_ref[f],
                        preferred_element_type=jnp.float32,
                        precision=lax.Precision.HIGHEST)
    dinv = _dinv_blk(d0_ref, d1_ref)
    out_ref[...] = jnp.dot(h, w_ref[...],
                           preferred_element_type=jnp.float32,
                        precision=lax.Precision.HIGHEST) * dinv


def _tcl_body(p0_ref, p1_ref, hs_ref, d0_ref, d1_ref, b_ref, w_ref, out_ref,
              *, relu):
    dinv = _dinv_blk(d0_ref, d1_ref)
    t = (p0_ref[0] + p1_ref[0] + hs_ref[...]) * dinv + b_ref[...]
    if relu:
        t = jnp.maximum(t, 0.0)
    out_ref[...] = jnp.dot(t, w_ref[...],
                           preferred_element_type=jnp.float32,
                        precision=lax.Precision.HIGHEST) * dinv


def _tc3_body(p0_ref, p1_ref, hs_ref, d0_ref, d1_ref, b_ref, out_ref):
    dinv = _dinv_blk(d0_ref, d1_ref)
    out_ref[...] = (p0_ref[0] + p1_ref[0] + hs_ref[...]) * dinv + b_ref[...]


def _tc4_body(pp_ref, cc_ref, lw_ref, lb_ref, out_ref):
    cnt = jnp.maximum(cc_ref[:, 0:1], 1.0)
    pooled = pp_ref[...] / cnt
    logit = jnp.sum(pooled * lw_ref[...], axis=1, keepdims=True) + lb_ref[0, 0]
    out_ref[...] = jax.nn.sigmoid(logit)


_row_spec = pl.BlockSpec((BLK, H), lambda i: (i, 0))
_part0_spec = pl.BlockSpec((1, BLK, H), lambda i: (0, i, 0))
_part1_spec = pl.BlockSpec((1, BLK, H), lambda i: (1, i, 0))
_deg0_spec = pl.BlockSpec((1, BLK, H), lambda i: (0, i, 0))
_deg1_spec = pl.BlockSpec((1, BLK, H), lambda i: (1, i, 0))
_w_spec = pl.BlockSpec((H, H), lambda i: (0, 0))
_b_spec = pl.BlockSpec((1, H), lambda i: (0, 0))
_node_out = jax.ShapeDtypeStruct((N, H), jnp.float32)


def _tc0(x, tables, deg, W1):
    return pl.pallas_call(
        _tc0_body,
        grid=(N // BLK,),
        in_specs=[
            pl.BlockSpec((BLK, NF), lambda i: (i, 0)),
            pl.BlockSpec((NF, V, H), lambda i: (0, 0, 0)),
            _deg0_spec, _deg1_spec, _w_spec,
        ],
        out_specs=_row_spec,
        out_shape=_node_out,
    )(x, tables, deg, deg, W1)


def _tcl(parts, hs, deg, b, Wn, relu):
    return pl.pallas_call(
        functools.partial(_tcl_body, relu=relu),
        grid=(N // BLK,),
        in_specs=[_part0_spec, _part1_spec, _row_spec, _deg0_spec, _deg1_spec,
                  _b_spec, _w_spec],
        out_specs=_row_spec,
        out_shape=_node_out,
    )(parts, parts, hs, deg, deg, b.reshape(1, H), Wn)


def _tc3(parts, hs, deg, b):
    return pl.pallas_call(
        _tc3_body,
        grid=(N // BLK,),
        in_specs=[_part0_spec, _part1_spec, _row_spec, _deg0_spec, _deg1_spec,
                  _b_spec],
        out_specs=_row_spec,
        out_shape=_node_out,
    )(parts, parts, hs, deg, deg, b.reshape(1, H))


def _tc4(pooled, counts, lin_w, lin_b):
    return pl.pallas_call(
        _tc4_body,
        in_specs=[
            pl.BlockSpec((G, H), lambda: (0, 0)),
            pl.BlockSpec((G, H), lambda: (0, 0)),
            pl.BlockSpec((1, H), lambda: (0, 0)),
            pl.BlockSpec((1, 1), lambda: (0, 0)),
        ],
        out_specs=pl.BlockSpec((G, 1), lambda: (0, 0)),
        out_shape=jax.ShapeDtypeStruct((G, 1), jnp.float32),
    )(pooled, counts, lin_w.reshape(1, H), lin_b.reshape(1, 1))


def kernel(x, edge_index, batch, tables, W1, b1, W2, b2, W3, b3, lin_w, lin_b):
    src = edge_index[0].astype(jnp.int32)
    dst = edge_index[1].astype(jnp.int32)
    npad = E_PAD - E
    src_pad = jnp.concatenate([src, jnp.zeros((npad,), jnp.int32)])
    dst_pad = jnp.concatenate([dst, jnp.full((npad,), N, jnp.int32)])

    deg = _deg_pass(dst_pad)

    hs1 = _tc0(x.astype(jnp.int32), tables, deg, W1)
    parts = _conv_pass(src_pad, dst_pad, hs1)
    hs2 = _tcl(parts, hs1, deg, b1, W2, relu=True)
    parts = _conv_pass(src_pad, dst_pad, hs2)
    hs3 = _tcl(parts, hs2, deg, b2, W3, relu=True)
    parts = _conv_pass(src_pad, dst_pad, hs3)
    h_out = _tc3(parts, hs3, deg, b3)

    pooled, counts = _pool_pass(h_out, batch.astype(jnp.int32))
    return _tc4(pooled, counts, lin_w, lin_b)


# speedup vs baseline: 7.3578x; 1.2061x over previous
import functools

import jax
import jax.numpy as jnp
from jax import lax
from jax.experimental import pallas as pl
from jax.experimental.pallas import tpu as pltpu
from jax.experimental.pallas import tpu_sc as plsc

N = 10000
E = 320000
H = 128
NF = 9
V = 64
G = 256

NC = 2
NS = 16
NW = NC * NS

CHUNK = 128
CH_PER_W = 80
SLAB = 40
EPW = CHUNK * CH_PER_W
E_PAD = NW * EPW
NP = 10240
RPT = NP // NS
IC = 128
GPT = G // NS
PC = 80
PNCH = N // PC

_mesh = plsc.VectorSubcoreMesh(core_axis_name="c", subcore_axis_name="s")


def _zero_rows(buf, nrows, ncol16):
    z16 = jnp.zeros((16,), jnp.float32)
    def body(r, _):
        for k in range(ncol16):
            buf[r, pl.ds(k * 16, 16)] = z16
        return 0
    lax.fori_loop(0, nrows, body, 0)


def _fill_ones(buf, nrows, ncol16):
    o16 = jnp.ones((16,), jnp.float32)
    def body(r, _):
        for k in range(ncol16):
            buf[r, pl.ds(k * 16, 16)] = o16
        return 0
    lax.fori_loop(0, nrows, body, 0)


@functools.partial(
    pl.kernel,
    out_type=jax.ShapeDtypeStruct((NC, NP, H), jnp.float32),
    mesh=_mesh,
    scratch_types=[
        pltpu.VMEM((CHUNK,), jnp.int32),
        pltpu.VMEM((CHUNK, H), jnp.float32),
        pltpu.VMEM_SHARED((NP, H), jnp.float32),
    ],
)
def _deg_pass(dst_hbm, deg_hbm, didx, obuf, deg2d):
    c = lax.axis_index("c")
    s = lax.axis_index("s")
    wid = c * NS + s

    _zero_rows(obuf, IC, H // 16)
    for j in range(RPT // IC):
        pltpu.sync_copy(obuf.at[pl.ds(0, IC)],
                        deg2d.at[pl.ds(s * RPT + j * IC, IC)])
    _fill_ones(obuf, CHUNK, H // 16)
    plsc.subcore_barrier()

    base0 = wid * EPW

    def chunk(i, _):
        eb = pl.multiple_of(base0 + i * CHUNK, CHUNK)
        pltpu.sync_copy(dst_hbm.at[pl.ds(eb, CHUNK)], didx)
        pltpu.sync_copy(obuf, deg2d.at[didx], add=True)
        return 0

    lax.fori_loop(0, CH_PER_W, chunk, 0)
    plsc.subcore_barrier()

    rb = s * RPT
    pltpu.sync_copy(deg2d.at[pl.ds(rb, RPT)], deg_hbm.at[c, pl.ds(rb, RPT)])


@functools.partial(
    pl.kernel,
    out_type=jax.ShapeDtypeStruct((NC, NP, H), jnp.float32),
    mesh=_mesh,
    scratch_types=[
        pltpu.VMEM((SLAB, CHUNK), jnp.int32),
        pltpu.VMEM((SLAB, CHUNK), jnp.int32),
        pltpu.VMEM((CHUNK, H), jnp.float32),
        pltpu.VMEM((CHUNK, H), jnp.float32),
        pltpu.VMEM_SHARED((NP, H), jnp.float32),
        pltpu.SemaphoreType.DMA,
        pltpu.SemaphoreType.DMA,
    ],
)
def _conv_pass(src_hbm, dst_hbm, hs_hbm, out_hbm, sidx2, didx2, rows_a, rows_b,
               acc, sem0, sem1):
    c = lax.axis_index("c")
    s = lax.axis_index("s")
    wid = c * NS + s

    pltpu.async_copy(src_hbm.at[wid, pl.ds(0, SLAB)], sidx2, sem0)
    pltpu.async_copy(dst_hbm.at[wid, pl.ds(0, SLAB)], didx2, sem1)

    _zero_rows(rows_a, CHUNK, H // 16)
    for j in range(RPT // IC):
        pltpu.sync_copy(rows_a,
                        acc.at[pl.ds(s * RPT + j * IC, IC)])
    pltpu.make_async_copy(src_hbm.at[wid, pl.ds(0, SLAB)], sidx2, sem0).wait()
    pltpu.make_async_copy(dst_hbm.at[wid, pl.ds(0, SLAB)], didx2, sem1).wait()
    plsc.subcore_barrier()

    sems = (sem0, sem1)
    bufs = (rows_a, rows_b)

    for half in range(CH_PER_W // SLAB):
        if half > 0:
            hb = pl.multiple_of(half * SLAB, 8)
            pltpu.sync_copy(src_hbm.at[wid, pl.ds(hb, SLAB)], sidx2)
            pltpu.sync_copy(dst_hbm.at[wid, pl.ds(hb, SLAB)], didx2)
        pltpu.async_copy(hs_hbm.at[sidx2.at[0]], rows_a, sem0)

        def outer(j, _):
            for b in range(2):
                ch = 2 * j + b
                @pl.when(ch < SLAB - 1)
                def _():
                    pltpu.async_copy(hs_hbm.at[sidx2.at[ch + 1]],
                                     bufs[1 - b], sems[1 - b])
                pltpu.make_async_copy(hs_hbm.at[sidx2.at[ch]],
                                      bufs[b], sems[b]).wait()
                pltpu.sync_copy(bufs[b], acc.at[didx2.at[ch]], add=True)
            return 0

        lax.fori_loop(0, SLAB // 2, outer, 0)

    plsc.subcore_barrier()

    rb = s * RPT
    pltpu.sync_copy(acc.at[pl.ds(rb, RPT)], out_hbm.at[c, pl.ds(rb, RPT)])


@functools.partial(
    pl.kernel,
    out_type=(jax.ShapeDtypeStruct((G, H), jnp.float32),
              jax.ShapeDtypeStruct((G, H), jnp.float32)),
    mesh=_mesh,
    scratch_types=[
        pltpu.VMEM((PC,), jnp.int32),
        pltpu.VMEM((PC, H), jnp.float32),
        pltpu.VMEM((PC, H), jnp.float32),
        pltpu.VMEM_SHARED((G, H), jnp.float32),
        pltpu.VMEM_SHARED((G, H), jnp.float32),
    ],
)
def _pool_pass(h_hbm, batch_hbm, pooled_hbm, counts_hbm,
               bidx, prow, cbuf, pooled2d, counts2d):
    c = lax.axis_index("c")
    s = lax.axis_index("s")
    nch = (PNCH - s + NS - 1) // NS

    @pl.when(c == 0)
    def _():
        _zero_rows(prow, GPT, H // 16)
        pltpu.sync_copy(prow.at[pl.ds(0, GPT)],
                        pooled2d.at[pl.ds(s * GPT, GPT)])
        plsc.subcore_barrier()

        def chunk(j, _):
            eb = pl.multiple_of((s + j * NS) * PC, PC)
            pltpu.sync_copy(batch_hbm.at[pl.ds(eb, PC)], bidx)
            pltpu.sync_copy(h_hbm.at[pl.ds(eb, PC)], prow)
            pltpu.sync_copy(prow, pooled2d.at[bidx], add=True)
            return 0

        lax.fori_loop(0, nch, chunk, 0)
        plsc.subcore_barrier()
        pltpu.sync_copy(pooled2d.at[pl.ds(s * GPT, GPT)],
                        pooled_hbm.at[pl.ds(s * GPT, GPT)])

    @pl.when(c == 1)
    def _():
        _zero_rows(cbuf, GPT, H // 16)
        pltpu.sync_copy(cbuf.at[pl.ds(0, GPT)],
                        counts2d.at[pl.ds(s * GPT, GPT)])
        _fill_ones(cbuf, PC, H // 16)
        plsc.subcore_barrier()

        def chunk(j, _):
            eb = pl.multiple_of((s + j * NS) * PC, PC)
            pltpu.sync_copy(batch_hbm.at[pl.ds(eb, PC)], bidx)
            pltpu.sync_copy(cbuf, counts2d.at[bidx], add=True)
            return 0

        lax.fori_loop(0, nch, chunk, 0)
        plsc.subcore_barrier()
        pltpu.sync_copy(counts2d.at[pl.ds(s * GPT, GPT)],
                        counts_hbm.at[pl.ds(s * GPT, GPT)])


BLK = 2000


def _dinv_blk(d0_ref, d1_ref):
    deg = d0_ref[0, :, 0:1] + d1_ref[0, :, 0:1] + 1.0
    return lax.rsqrt(deg)


def _tc0_body(x_ref, tab_ref, d0_ref, d1_ref, w_ref, out_ref):
    h = jnp.zeros((BLK, H), jnp.float32)
    for f in range(NF):
        xf = x_ref[:, f:f + 1]
        oh = (xf == lax.broadcasted_iota(jnp.int32, (BLK, V), 1))
        h = h + jnp.dot(oh.astype(jnp.float32), tab_ref[f],
                        preferred_element_type=jnp.float32,
                        precision=lax.Precision.HIGHEST)
    dinv = _dinv_blk(d0_ref, d1_ref)
    out_ref[...] = jnp.dot(h, w_ref[...],
                           preferred_element_type=jnp.float32,
                        precision=lax.Precision.HIGHEST) * dinv


def _tcl_body(p0_ref, p1_ref, hs_ref, d0_ref, d1_ref, b_ref, w_ref, out_ref,
              *, relu):
    dinv = _dinv_blk(d0_ref, d1_ref)
    t = (p0_ref[0] + p1_ref[0] + hs_ref[...]) * dinv + b_ref[...]
    if relu:
        t = jnp.maximum(t, 0.0)
    out_ref[...] = jnp.dot(t, w_ref[...],
                           preferred_element_type=jnp.float32,
                        precision=lax.Precision.HIGHEST) * dinv


def _tc3_body(p0_ref, p1_ref, hs_ref, d0_ref, d1_ref, b_ref, out_ref):
    dinv = _dinv_blk(d0_ref, d1_ref)
    out_ref[...] = (p0_ref[0] + p1_ref[0] + hs_ref[...]) * dinv + b_ref[...]


def _tc4_body(pp_ref, cc_ref, lw_ref, lb_ref, out_ref):
    cnt = jnp.maximum(cc_ref[:, 0:1], 1.0)
    pooled = pp_ref[...] / cnt
    logit = jnp.sum(pooled * lw_ref[...], axis=1, keepdims=True) + lb_ref[0, 0]
    out_ref[...] = jax.nn.sigmoid(logit)


_row_spec = pl.BlockSpec((BLK, H), lambda i: (i, 0))
_part0_spec = pl.BlockSpec((1, BLK, H), lambda i: (0, i, 0))
_part1_spec = pl.BlockSpec((1, BLK, H), lambda i: (1, i, 0))
_deg0_spec = pl.BlockSpec((1, BLK, H), lambda i: (0, i, 0))
_deg1_spec = pl.BlockSpec((1, BLK, H), lambda i: (1, i, 0))
_w_spec = pl.BlockSpec((H, H), lambda i: (0, 0))
_b_spec = pl.BlockSpec((1, H), lambda i: (0, 0))
_node_out = jax.ShapeDtypeStruct((N, H), jnp.float32)


def _tc0(x, tables, deg, W1):
    return pl.pallas_call(
        _tc0_body,
        grid=(N // BLK,),
        in_specs=[
            pl.BlockSpec((BLK, NF), lambda i: (i, 0)),
            pl.BlockSpec((NF, V, H), lambda i: (0, 0, 0)),
            _deg0_spec, _deg1_spec, _w_spec,
        ],
        out_specs=_row_spec,
        out_shape=_node_out,
    )(x, tables, deg, deg, W1)


def _tcl(parts, hs, deg, b, Wn, relu):
    return pl.pallas_call(
        functools.partial(_tcl_body, relu=relu),
        grid=(N // BLK,),
        in_specs=[_part0_spec, _part1_spec, _row_spec, _deg0_spec, _deg1_spec,
                  _b_spec, _w_spec],
        out_specs=_row_spec,
        out_shape=_node_out,
    )(parts, parts, hs, deg, deg, b.reshape(1, H), Wn)


def _tc3(parts, hs, deg, b):
    return pl.pallas_call(
        _tc3_body,
        grid=(N // BLK,),
        in_specs=[_part0_spec, _part1_spec, _row_spec, _deg0_spec, _deg1_spec,
                  _b_spec],
        out_specs=_row_spec,
        out_shape=_node_out,
    )(parts, parts, hs, deg, deg, b.reshape(1, H))


def _tc4(pooled, counts, lin_w, lin_b):
    return pl.pallas_call(
        _tc4_body,
        in_specs=[
            pl.BlockSpec((G, H), lambda: (0, 0)),
            pl.BlockSpec((G, H), lambda: (0, 0)),
            pl.BlockSpec((1, H), lambda: (0, 0)),
            pl.BlockSpec((1, 1), lambda: (0, 0)),
        ],
        out_specs=pl.BlockSpec((G, 1), lambda: (0, 0)),
        out_shape=jax.ShapeDtypeStruct((G, 1), jnp.float32),
    )(pooled, counts, lin_w.reshape(1, H), lin_b.reshape(1, 1))


def kernel(x, edge_index, batch, tables, W1, b1, W2, b2, W3, b3, lin_w, lin_b):
    src = edge_index[0].astype(jnp.int32)
    dst = edge_index[1].astype(jnp.int32)
    npad = E_PAD - E
    src_pad = jnp.concatenate([src, jnp.zeros((npad,), jnp.int32)])
    dst_pad = jnp.concatenate(
        [dst, N + jnp.arange(npad, dtype=jnp.int32) % (NP - N)])
    src3 = src_pad.reshape(NW, CH_PER_W, CHUNK)
    dst3 = dst_pad.reshape(NW, CH_PER_W, CHUNK)

    deg = _deg_pass(dst_pad)

    hs1 = _tc0(x.astype(jnp.int32), tables, deg, W1)
    parts = _conv_pass(src3, dst3, hs1)
    hs2 = _tcl(parts, hs1, deg, b1, W2, relu=True)
    parts = _conv_pass(src3, dst3, hs2)
    hs3 = _tcl(parts, hs2, deg, b2, W3, relu=True)
    parts = _conv_pass(src3, dst3, hs3)
    h_out = _tc3(parts, hs3, deg, b3)

    pooled, counts = _pool_pass(h_out, batch.astype(jnp.int32))
    return _tc4(pooled, counts, lin_w, lin_b)


# speedup vs baseline: 22.4629x; 3.0529x over previous
import functools

import jax
import jax.numpy as jnp
from jax import lax
from jax.experimental import pallas as pl
from jax.experimental.pallas import tpu as pltpu
from jax.experimental.pallas import tpu_sc as plsc

N = 10000
E = 320000
H = 128
NF = 9
V = 64
G = 256

NC = 2
NS = 16
NW = NC * NS

CHUNK = 128
CH_PER_W = 80
SLAB = 40
EPW = CHUNK * CH_PER_W
E_PAD = NW * EPW
NP = 10240
RPT = NP // NS
IC = 128
GPT = G // NS
PC = 80
PNCH = N // PC

_mesh = plsc.VectorSubcoreMesh(core_axis_name="c", subcore_axis_name="s")


def _zero_rows(buf, nrows, ncol16):
    z16 = jnp.zeros((16,), jnp.float32)
    def body(r, _):
        for k in range(ncol16):
            buf[r, pl.ds(k * 16, 16)] = z16
        return 0
    lax.fori_loop(0, nrows, body, 0)


def _fill_ones(buf, nrows, ncol16):
    o16 = jnp.ones((16,), jnp.float32)
    def body(r, _):
        for k in range(ncol16):
            buf[r, pl.ds(k * 16, 16)] = o16
        return 0
    lax.fori_loop(0, nrows, body, 0)


@functools.partial(
    pl.kernel,
    out_type=jax.ShapeDtypeStruct((NC, NP, H), jnp.float32),
    mesh=_mesh,
    scratch_types=[
        pltpu.VMEM((CHUNK,), jnp.int32),
        pltpu.VMEM((CHUNK, H), jnp.float32),
        pltpu.VMEM_SHARED((NP, H), jnp.float32),
    ],
)
def _deg_pass(dst_hbm, deg_hbm, didx, obuf, deg2d):
    c = lax.axis_index("c")
    s = lax.axis_index("s")
    wid = c * NS + s

    _zero_rows(obuf, IC, H // 16)
    for j in range(RPT // IC):
        pltpu.sync_copy(obuf.at[pl.ds(0, IC)],
                        deg2d.at[pl.ds(s * RPT + j * IC, IC)])
    _fill_ones(obuf, CHUNK, H // 16)
    plsc.subcore_barrier()

    base0 = wid * EPW

    def chunk(i, _):
        eb = pl.multiple_of(base0 + i * CHUNK, CHUNK)
        pltpu.sync_copy(dst_hbm.at[pl.ds(eb, CHUNK)], didx)
        pltpu.sync_copy(obuf, deg2d.at[didx], add=True)
        return 0

    lax.fori_loop(0, CH_PER_W, chunk, 0)
    plsc.subcore_barrier()

    rb = s * RPT
    pltpu.sync_copy(deg2d.at[pl.ds(rb, RPT)], deg_hbm.at[c, pl.ds(rb, RPT)])


@functools.partial(
    pl.kernel,
    out_type=jax.ShapeDtypeStruct((NC, NP, H), jnp.float32),
    mesh=_mesh,
    scratch_types=[
        pltpu.VMEM((SLAB, CHUNK), jnp.int32),
        pltpu.VMEM((SLAB, CHUNK), jnp.int32),
        pltpu.VMEM((CHUNK, H), jnp.float32),
        pltpu.VMEM((CHUNK, H), jnp.float32),
        pltpu.VMEM_SHARED((NP, H), jnp.float32),
        pltpu.SemaphoreType.DMA,
        pltpu.SemaphoreType.DMA,
    ],
)
def _conv_pass(src_hbm, dst_hbm, hs_hbm, out_hbm, sidx2, didx2, rows_a, rows_b,
               acc, sem0, sem1):
    c = lax.axis_index("c")
    s = lax.axis_index("s")
    wid = c * NS + s

    pltpu.async_copy(src_hbm.at[wid, pl.ds(0, SLAB)], sidx2, sem0)
    pltpu.async_copy(dst_hbm.at[wid, pl.ds(0, SLAB)], didx2, sem1)

    _zero_rows(rows_a, CHUNK, H // 16)
    for j in range(RPT // IC):
        pltpu.sync_copy(rows_a,
                        acc.at[pl.ds(s * RPT + j * IC, IC)])
    pltpu.make_async_copy(src_hbm.at[wid, pl.ds(0, SLAB)], sidx2, sem0).wait()
    pltpu.make_async_copy(dst_hbm.at[wid, pl.ds(0, SLAB)], didx2, sem1).wait()
    plsc.subcore_barrier()

    sems = (sem0, sem1)
    bufs = (rows_a, rows_b)

    for half in range(CH_PER_W // SLAB):
        if half > 0:
            hb = pl.multiple_of(half * SLAB, 8)
            pltpu.sync_copy(src_hbm.at[wid, pl.ds(hb, SLAB)], sidx2)
            pltpu.sync_copy(dst_hbm.at[wid, pl.ds(hb, SLAB)], didx2)
        pltpu.async_copy(hs_hbm.at[sidx2.at[0]], rows_a, sem0)

        def outer(j, _):
            for b in range(2):
                ch = 2 * j + b
                @pl.when(ch < SLAB - 1)
                def _():
                    pltpu.async_copy(hs_hbm.at[sidx2.at[ch + 1]],
                                     bufs[1 - b], sems[1 - b])
                pltpu.make_async_copy(hs_hbm.at[sidx2.at[ch]],
                                      bufs[b], sems[b]).wait()
                pltpu.sync_copy(bufs[b], acc.at[didx2.at[ch]], add=True)
            return 0

        lax.fori_loop(0, SLAB // 2, outer, 0)

    plsc.subcore_barrier()

    rb = s * RPT
    pltpu.sync_copy(acc.at[pl.ds(rb, RPT)], out_hbm.at[c, pl.ds(rb, RPT)])


@functools.partial(
    pl.kernel,
    out_type=(jax.ShapeDtypeStruct((G, H), jnp.float32),
              jax.ShapeDtypeStruct((G, H), jnp.float32)),
    mesh=_mesh,
    scratch_types=[
        pltpu.VMEM((PC,), jnp.int32),
        pltpu.VMEM((PC, H), jnp.float32),
        pltpu.VMEM((PC, H), jnp.float32),
        pltpu.VMEM_SHARED((G, H), jnp.float32),
        pltpu.VMEM_SHARED((G, H), jnp.float32),
    ],
)
def _pool_pass(h_hbm, batch_hbm, pooled_hbm, counts_hbm,
               bidx, prow, cbuf, pooled2d, counts2d):
    c = lax.axis_index("c")
    s = lax.axis_index("s")
    nch = (PNCH - s + NS - 1) // NS

    @pl.when(c == 0)
    def _():
        _zero_rows(prow, GPT, H // 16)
        pltpu.sync_copy(prow.at[pl.ds(0, GPT)],
                        pooled2d.at[pl.ds(s * GPT, GPT)])
        plsc.subcore_barrier()

        def chunk(j, _):
            eb = pl.multiple_of((s + j * NS) * PC, PC)
            pltpu.sync_copy(batch_hbm.at[pl.ds(eb, PC)], bidx)
            pltpu.sync_copy(h_hbm.at[pl.ds(eb, PC)], prow)
            pltpu.sync_copy(prow, pooled2d.at[bidx], add=True)
            return 0

        lax.fori_loop(0, nch, chunk, 0)
        plsc.subcore_barrier()
        pltpu.sync_copy(pooled2d.at[pl.ds(s * GPT, GPT)],
                        pooled_hbm.at[pl.ds(s * GPT, GPT)])

    @pl.when(c == 1)
    def _():
        _zero_rows(cbuf, GPT, H // 16)
        pltpu.sync_copy(cbuf.at[pl.ds(0, GPT)],
                        counts2d.at[pl.ds(s * GPT, GPT)])
        _fill_ones(cbuf, PC, H // 16)
        plsc.subcore_barrier()

        def chunk(j, _):
            eb = pl.multiple_of((s + j * NS) * PC, PC)
            pltpu.sync_copy(batch_hbm.at[pl.ds(eb, PC)], bidx)
            pltpu.sync_copy(cbuf, counts2d.at[bidx], add=True)
            return 0

        lax.fori_loop(0, nch, chunk, 0)
        plsc.subcore_barrier()
        pltpu.sync_copy(counts2d.at[pl.ds(s * GPT, GPT)],
                        counts_hbm.at[pl.ds(s * GPT, GPT)])


BLK = 2000


def _dinv_blk(d0_ref, d1_ref):
    deg = d0_ref[0, :, 0:1] + d1_ref[0, :, 0:1] + 1.0
    return lax.rsqrt(deg)


def _tc0_body(x_ref, tab_ref, d0_ref, d1_ref, w_ref, out_ref):
    h = jnp.zeros((BLK, H), jnp.float32)
    for f in range(NF):
        xf = x_ref[:, f:f + 1]
        oh = (xf == lax.broadcasted_iota(jnp.int32, (BLK, V), 1))
        h = h + jnp.dot(oh.astype(jnp.float32), tab_ref[f],
                        preferred_element_type=jnp.float32,
                        precision=lax.Precision.HIGHEST)
    dinv = _dinv_blk(d0_ref, d1_ref)
    out_ref[...] = jnp.dot(h, w_ref[...],
                           preferred_element_type=jnp.float32,
                        precision=lax.Precision.HIGHEST) * dinv


def _tcl_body(p0_ref, p1_ref, hs_ref, d0_ref, d1_ref, b_ref, w_ref, out_ref,
              *, relu):
    dinv = _dinv_blk(d0_ref, d1_ref)
    t = (p0_ref[0] + p1_ref[0] + hs_ref[...]) * dinv + b_ref[...]
    if relu:
        t = jnp.maximum(t, 0.0)
    out_ref[...] = jnp.dot(t, w_ref[...],
                           preferred_element_type=jnp.float32,
                        precision=lax.Precision.HIGHEST) * dinv


def _tc3_body(p0_ref, p1_ref, hs_ref, d0_ref, d1_ref, b_ref, out_ref):
    dinv = _dinv_blk(d0_ref, d1_ref)
    out_ref[...] = (p0_ref[0] + p1_ref[0] + hs_ref[...]) * dinv + b_ref[...]


def _tc4_body(pp_ref, cc_ref, lw_ref, lb_ref, out_ref):
    cnt = jnp.maximum(cc_ref[:, 0:1], 1.0)
    pooled = pp_ref[...] / cnt
    logit = jnp.sum(pooled * lw_ref[...], axis=1, keepdims=True) + lb_ref[0, 0]
    out_ref[...] = jax.nn.sigmoid(logit)


_row_spec = pl.BlockSpec((BLK, H), lambda i: (i, 0))
_part0_spec = pl.BlockSpec((1, BLK, H), lambda i: (0, i, 0))
_part1_spec = pl.BlockSpec((1, BLK, H), lambda i: (1, i, 0))
_deg0_spec = pl.BlockSpec((1, BLK, H), lambda i: (0, i, 0))
_deg1_spec = pl.BlockSpec((1, BLK, H), lambda i: (1, i, 0))
_w_spec = pl.BlockSpec((H, H), lambda i: (0, 0))
_b_spec = pl.BlockSpec((1, H), lambda i: (0, 0))
_node_out = jax.ShapeDtypeStruct((N, H), jnp.float32)


def _tc0(x, tables, deg, W1):
    return pl.pallas_call(
        _tc0_body,
        grid=(N // BLK,),
        in_specs=[
            pl.BlockSpec((BLK, NF), lambda i: (i, 0)),
            pl.BlockSpec((NF, V, H), lambda i: (0, 0, 0)),
            _deg0_spec, _deg1_spec, _w_spec,
        ],
        out_specs=_row_spec,
        out_shape=_node_out,
    )(x, tables, deg, deg, W1)


def _tcl(parts, hs, deg, b, Wn, relu):
    return pl.pallas_call(
        functools.partial(_tcl_body, relu=relu),
        grid=(N // BLK,),
        in_specs=[_part0_spec, _part1_spec, _row_spec, _deg0_spec, _deg1_spec,
                  _b_spec, _w_spec],
        out_specs=_row_spec,
        out_shape=_node_out,
    )(parts, parts, hs, deg, deg, b.reshape(1, H), Wn)


def _tc3(parts, hs, deg, b):
    return pl.pallas_call(
        _tc3_body,
        grid=(N // BLK,),
        in_specs=[_part0_spec, _part1_spec, _row_spec, _deg0_spec, _deg1_spec,
                  _b_spec],
        out_specs=_row_spec,
        out_shape=_node_out,
    )(parts, parts, hs, deg, deg, b.reshape(1, H))


def _tc4(pooled, counts, lin_w, lin_b):
    return pl.pallas_call(
        _tc4_body,
        in_specs=[
            pl.BlockSpec((G, H), lambda: (0, 0)),
            pl.BlockSpec((G, H), lambda: (0, 0)),
            pl.BlockSpec((1, H), lambda: (0, 0)),
            pl.BlockSpec((1, 1), lambda: (0, 0)),
        ],
        out_specs=pl.BlockSpec((G, 1), lambda: (0, 0)),
        out_shape=jax.ShapeDtypeStruct((G, 1), jnp.float32),
    )(pooled, counts, lin_w.reshape(1, H), lin_b.reshape(1, 1))


def kernel(x, edge_index, batch, tables, W1, b1, W2, b2, W3, b3, lin_w, lin_b):
    src = edge_index[0].astype(jnp.int32)
    dst = edge_index[1].astype(jnp.int32)
    npad = E_PAD - E
    src_pad = jnp.concatenate(
        [src, jnp.arange(npad, dtype=jnp.int32) * 37 % N])
    dst_pad = jnp.concatenate(
        [dst, N + jnp.arange(npad, dtype=jnp.int32) % (NP - N)])
    src3 = src_pad.reshape(NW, CH_PER_W, CHUNK)
    dst3 = dst_pad.reshape(NW, CH_PER_W, CHUNK)

    deg = _deg_pass(dst_pad)

    hs1 = _tc0(x.astype(jnp.int32), tables, deg, W1)
    parts = _conv_pass(src3, dst3, hs1)
    hs2 = _tcl(parts, hs1, deg, b1, W2, relu=True)
    parts = _conv_pass(src3, dst3, hs2)
    hs3 = _tcl(parts, hs2, deg, b2, W3, relu=True)
    parts = _conv_pass(src3, dst3, hs3)
    h_out = _tc3(parts, hs3, deg, b3)

    pooled, counts = _pool_pass(h_out, batch.astype(jnp.int32))
    return _tc4(pooled, counts, lin_w, lin_b)


# speedup vs baseline: 24.1229x; 1.0739x over previous
import functools

import jax
import jax.numpy as jnp
from jax import lax
from jax.experimental import pallas as pl
from jax.experimental.pallas import tpu as pltpu
from jax.experimental.pallas import tpu_sc as plsc

N = 10000
E = 320000
H = 128
NF = 9
V = 64
G = 256

NC = 2
NS = 16
NW = NC * NS

CHUNK = 128
CH_PER_W = 80
SLAB = 40
EPW = CHUNK * CH_PER_W
E_PAD = NW * EPW
NP = 10240
RPT = NP // NS
IC = 128
GPT = G // NS
PC = 80
PNCH = N // PC

_mesh = plsc.VectorSubcoreMesh(core_axis_name="c", subcore_axis_name="s")


def _zero_rows(buf, nrows, ncol16):
    z16 = jnp.zeros((16,), jnp.float32)
    def body(r, _):
        for k in range(ncol16):
            buf[r, pl.ds(k * 16, 16)] = z16
        return 0
    lax.fori_loop(0, nrows, body, 0)


def _fill_ones(buf, nrows, ncol16):
    o16 = jnp.ones((16,), jnp.float32)
    def body(r, _):
        for k in range(ncol16):
            buf[r, pl.ds(k * 16, 16)] = o16
        return 0
    lax.fori_loop(0, nrows, body, 0)


@functools.partial(
    pl.kernel,
    out_type=jax.ShapeDtypeStruct((NC, NP, H), jnp.float32),
    mesh=_mesh,
    scratch_types=[
        pltpu.VMEM((CHUNK,), jnp.int32),
        pltpu.VMEM((CHUNK, H), jnp.float32),
        pltpu.VMEM_SHARED((NP, H), jnp.float32),
    ],
)
def _deg_pass(dst_hbm, deg_hbm, didx, obuf, deg2d):
    c = lax.axis_index("c")
    s = lax.axis_index("s")
    wid = c * NS + s

    _zero_rows(obuf, IC, H // 16)
    for j in range(RPT // IC):
        pltpu.sync_copy(obuf.at[pl.ds(0, IC)],
                        deg2d.at[pl.ds(s * RPT + j * IC, IC)])
    _fill_ones(obuf, CHUNK, H // 16)
    plsc.subcore_barrier()

    base0 = wid * EPW

    def chunk(i, _):
        eb = pl.multiple_of(base0 + i * CHUNK, CHUNK)
        pltpu.sync_copy(dst_hbm.at[pl.ds(eb, CHUNK)], didx)
        pltpu.sync_copy(obuf, deg2d.at[didx], add=True)
        return 0

    lax.fori_loop(0, CH_PER_W, chunk, 0)
    plsc.subcore_barrier()

    rb = s * RPT
    pltpu.sync_copy(deg2d.at[pl.ds(rb, RPT)], deg_hbm.at[c, pl.ds(rb, RPT)])


@functools.partial(
    pl.kernel,
    out_type=jax.ShapeDtypeStruct((NC, NP, H), jnp.float32),
    mesh=_mesh,
    scratch_types=[
        pltpu.VMEM((SLAB, CHUNK), jnp.int32),
        pltpu.VMEM((SLAB, CHUNK), jnp.int32),
        pltpu.VMEM((CHUNK, H), jnp.float32),
        pltpu.VMEM((CHUNK, H), jnp.float32),
        pltpu.VMEM_SHARED((NP, H), jnp.float32),
        pltpu.SemaphoreType.DMA,
        pltpu.SemaphoreType.DMA,
    ],
)
def _conv_pass(src_hbm, dst_hbm, hs_hbm, out_hbm, sidx2, didx2, rows_a, rows_b,
               acc, sem0, sem1):
    c = lax.axis_index("c")
    s = lax.axis_index("s")
    wid = c * NS + s

    pltpu.async_copy(src_hbm.at[wid, pl.ds(0, SLAB)], sidx2, sem0)
    pltpu.async_copy(dst_hbm.at[wid, pl.ds(0, SLAB)], didx2, sem1)

    _zero_rows(rows_a, CHUNK, H // 16)
    for j in range(RPT // IC):
        pltpu.sync_copy(rows_a,
                        acc.at[pl.ds(s * RPT + j * IC, IC)])
    pltpu.make_async_copy(src_hbm.at[wid, pl.ds(0, SLAB)], sidx2, sem0).wait()
    pltpu.make_async_copy(dst_hbm.at[wid, pl.ds(0, SLAB)], didx2, sem1).wait()
    plsc.subcore_barrier()

    sems = (sem0, sem1)
    bufs = (rows_a, rows_b)

    for half in range(CH_PER_W // SLAB):
        if half > 0:
            hb = pl.multiple_of(half * SLAB, 8)
            pltpu.sync_copy(src_hbm.at[wid, pl.ds(hb, SLAB)], sidx2)
            pltpu.sync_copy(dst_hbm.at[wid, pl.ds(hb, SLAB)], didx2)
        pltpu.async_copy(hs_hbm.at[sidx2.at[0]], rows_a, sem0)

        def outer(j, _):
            for b in range(2):
                ch = 2 * j + b
                @pl.when(ch < SLAB - 1)
                def _():
                    pltpu.async_copy(hs_hbm.at[sidx2.at[ch + 1]],
                                     bufs[1 - b], sems[1 - b])
                pltpu.make_async_copy(hs_hbm.at[sidx2.at[ch]],
                                      bufs[b], sems[b]).wait()
                pltpu.sync_copy(bufs[b], acc.at[didx2.at[ch]], add=True)
            return 0

        lax.fori_loop(0, SLAB // 2, outer, 0)

    plsc.subcore_barrier()

    rb = s * RPT
    pltpu.sync_copy(acc.at[pl.ds(rb, RPT)], out_hbm.at[c, pl.ds(rb, RPT)])


@functools.partial(
    pl.kernel,
    out_type=(jax.ShapeDtypeStruct((G, H), jnp.float32),
              jax.ShapeDtypeStruct((G, H), jnp.float32)),
    mesh=_mesh,
    scratch_types=[
        pltpu.VMEM((PC,), jnp.int32),
        pltpu.VMEM((PC, H), jnp.float32),
        pltpu.VMEM((PC, H), jnp.float32),
        pltpu.VMEM_SHARED((G, H), jnp.float32),
        pltpu.VMEM_SHARED((G, H), jnp.float32),
    ],
)
def _pool_pass(h_hbm, batch_hbm, pooled_hbm, counts_hbm,
               bidx, prow, cbuf, pooled2d, counts2d):
    c = lax.axis_index("c")
    s = lax.axis_index("s")
    nch = (PNCH - s + NS - 1) // NS

    @pl.when(c == 0)
    def _():
        _zero_rows(prow, GPT, H // 16)
        pltpu.sync_copy(prow.at[pl.ds(0, GPT)],
                        pooled2d.at[pl.ds(s * GPT, GPT)])
        plsc.subcore_barrier()

        def chunk(j, _):
            eb = pl.multiple_of((s + j * NS) * PC, PC)
            pltpu.sync_copy(batch_hbm.at[pl.ds(eb, PC)], bidx)
            pltpu.sync_copy(h_hbm.at[pl.ds(eb, PC)], prow)
            pltpu.sync_copy(prow, pooled2d.at[bidx], add=True)
            return 0

        lax.fori_loop(0, nch, chunk, 0)
        plsc.subcore_barrier()
        pltpu.sync_copy(pooled2d.at[pl.ds(s * GPT, GPT)],
                        pooled_hbm.at[pl.ds(s * GPT, GPT)])

    @pl.when(c == 1)
    def _():
        _zero_rows(cbuf, GPT, H // 16)
        pltpu.sync_copy(cbuf.at[pl.ds(0, GPT)],
                        counts2d.at[pl.ds(s * GPT, GPT)])
        _fill_ones(cbuf, PC, H // 16)
        plsc.subcore_barrier()

        def chunk(j, _):
            eb = pl.multiple_of((s + j * NS) * PC, PC)
            pltpu.sync_copy(batch_hbm.at[pl.ds(eb, PC)], bidx)
            pltpu.sync_copy(cbuf, counts2d.at[bidx], add=True)
            return 0

        lax.fori_loop(0, nch, chunk, 0)
        plsc.subcore_barrier()
        pltpu.sync_copy(counts2d.at[pl.ds(s * GPT, GPT)],
                        counts_hbm.at[pl.ds(s * GPT, GPT)])


BLK = 2000


def _dinv_blk(d0_ref, d1_ref):
    deg = d0_ref[0, :, 0:1] + d1_ref[0, :, 0:1] + 1.0
    return lax.rsqrt(deg)


def _tc0_body(x_ref, tab_ref, d0_ref, d1_ref, w_ref, out_ref):
    h = jnp.zeros((BLK, H), jnp.float32)
    for f in range(NF):
        xf = x_ref[:, f:f + 1]
        oh = (xf == lax.broadcasted_iota(jnp.int32, (BLK, V), 1))
        h = h + jnp.dot(oh.astype(jnp.float32), tab_ref[f],
                        preferred_element_type=jnp.float32)
    dinv = _dinv_blk(d0_ref, d1_ref)
    out_ref[...] = jnp.dot(h, w_ref[...],
                           preferred_element_type=jnp.float32) * dinv


def _tcl_body(p0_ref, p1_ref, hs_ref, d0_ref, d1_ref, b_ref, w_ref, out_ref,
              *, relu):
    dinv = _dinv_blk(d0_ref, d1_ref)
    t = (p0_ref[0] + p1_ref[0] + hs_ref[...]) * dinv + b_ref[...]
    if relu:
        t = jnp.maximum(t, 0.0)
    out_ref[...] = jnp.dot(t, w_ref[...],
                           preferred_element_type=jnp.float32) * dinv


def _tc3_body(p0_ref, p1_ref, hs_ref, d0_ref, d1_ref, b_ref, out_ref):
    dinv = _dinv_blk(d0_ref, d1_ref)
    out_ref[...] = (p0_ref[0] + p1_ref[0] + hs_ref[...]) * dinv + b_ref[...]


def _tc4_body(pp_ref, cc_ref, lw_ref, lb_ref, out_ref):
    cnt = jnp.maximum(cc_ref[:, 0:1], 1.0)
    pooled = pp_ref[...] / cnt
    logit = jnp.sum(pooled * lw_ref[...], axis=1, keepdims=True) + lb_ref[0, 0]
    out_ref[...] = jax.nn.sigmoid(logit)


_row_spec = pl.BlockSpec((BLK, H), lambda i: (i, 0))
_part0_spec = pl.BlockSpec((1, BLK, H), lambda i: (0, i, 0))
_part1_spec = pl.BlockSpec((1, BLK, H), lambda i: (1, i, 0))
_deg0_spec = pl.BlockSpec((1, BLK, H), lambda i: (0, i, 0))
_deg1_spec = pl.BlockSpec((1, BLK, H), lambda i: (1, i, 0))
_w_spec = pl.BlockSpec((H, H), lambda i: (0, 0))
_b_spec = pl.BlockSpec((1, H), lambda i: (0, 0))
_node_out = jax.ShapeDtypeStruct((N, H), jnp.float32)


def _tc0(x, tables, deg, W1):
    return pl.pallas_call(
        _tc0_body,
        grid=(N // BLK,),
        in_specs=[
            pl.BlockSpec((BLK, NF), lambda i: (i, 0)),
            pl.BlockSpec((NF, V, H), lambda i: (0, 0, 0)),
            _deg0_spec, _deg1_spec, _w_spec,
        ],
        out_specs=_row_spec,
        out_shape=_node_out,
    )(x, tables, deg, deg, W1)


def _tcl(parts, hs, deg, b, Wn, relu):
    return pl.pallas_call(
        functools.partial(_tcl_body, relu=relu),
        grid=(N // BLK,),
        in_specs=[_part0_spec, _part1_spec, _row_spec, _deg0_spec, _deg1_spec,
                  _b_spec, _w_spec],
        out_specs=_row_spec,
        out_shape=_node_out,
    )(parts, parts, hs, deg, deg, b.reshape(1, H), Wn)


def _tc3(parts, hs, deg, b):
    return pl.pallas_call(
        _tc3_body,
        grid=(N // BLK,),
        in_specs=[_part0_spec, _part1_spec, _row_spec, _deg0_spec, _deg1_spec,
                  _b_spec],
        out_specs=_row_spec,
        out_shape=_node_out,
    )(parts, parts, hs, deg, deg, b.reshape(1, H))


def _tc4(pooled, counts, lin_w, lin_b):
    return pl.pallas_call(
        _tc4_body,
        in_specs=[
            pl.BlockSpec((G, H), lambda: (0, 0)),
            pl.BlockSpec((G, H), lambda: (0, 0)),
            pl.BlockSpec((1, H), lambda: (0, 0)),
            pl.BlockSpec((1, 1), lambda: (0, 0)),
        ],
        out_specs=pl.BlockSpec((G, 1), lambda: (0, 0)),
        out_shape=jax.ShapeDtypeStruct((G, 1), jnp.float32),
    )(pooled, counts, lin_w.reshape(1, H), lin_b.reshape(1, 1))


def kernel(x, edge_index, batch, tables, W1, b1, W2, b2, W3, b3, lin_w, lin_b):
    src = edge_index[0].astype(jnp.int32)
    dst = edge_index[1].astype(jnp.int32)
    npad = E_PAD - E
    src_pad = jnp.concatenate(
        [src, jnp.arange(npad, dtype=jnp.int32) * 37 % N])
    dst_pad = jnp.concatenate(
        [dst, N + jnp.arange(npad, dtype=jnp.int32) % (NP - N)])
    src3 = src_pad.reshape(NW, CH_PER_W, CHUNK)
    dst3 = dst_pad.reshape(NW, CH_PER_W, CHUNK)

    deg = _deg_pass(dst_pad)

    hs1 = _tc0(x.astype(jnp.int32), tables, deg, W1)
    parts = _conv_pass(src3, dst3, hs1)
    hs2 = _tcl(parts, hs1, deg, b1, W2, relu=True)
    parts = _conv_pass(src3, dst3, hs2)
    hs3 = _tcl(parts, hs2, deg, b2, W3, relu=True)
    parts = _conv_pass(src3, dst3, hs3)
    h_out = _tc3(parts, hs3, deg, b3)

    pooled, counts = _pool_pass(h_out, batch.astype(jnp.int32))
    return _tc4(pooled, counts, lin_w, lin_b)


# speedup vs baseline: 24.4209x; 1.0124x over previous
import functools

import jax
import jax.numpy as jnp
from jax import lax
from jax.experimental import pallas as pl
from jax.experimental.pallas import tpu as pltpu
from jax.experimental.pallas import tpu_sc as plsc

N = 10000
E = 320000
H = 128
NF = 9
V = 64
G = 256

NC = 2
NS = 16
NW = NC * NS

CHUNK = 128
CH_PER_W = 80
SLAB = 40
EPW = CHUNK * CH_PER_W
E_PAD = NW * EPW
NP = 10240
RPT = NP // NS
IC = 128
GPT = G // NS
PC = 80
PNCH = N // PC

_mesh = plsc.VectorSubcoreMesh(core_axis_name="c", subcore_axis_name="s")


def _zero_rows(buf, nrows, ncol16):
    z16 = jnp.zeros((16,), jnp.float32)
    def body(r, _):
        for k in range(ncol16):
            buf[r, pl.ds(k * 16, 16)] = z16
        return 0
    lax.fori_loop(0, nrows, body, 0)


def _fill_ones(buf, nrows, ncol16):
    o16 = jnp.ones((16,), jnp.float32)
    def body(r, _):
        for k in range(ncol16):
            buf[r, pl.ds(k * 16, 16)] = o16
        return 0
    lax.fori_loop(0, nrows, body, 0)


@functools.partial(
    pl.kernel,
    out_type=jax.ShapeDtypeStruct((NC, NP, H), jnp.float32),
    mesh=_mesh,
    scratch_types=[
        pltpu.VMEM((CHUNK,), jnp.int32),
        pltpu.VMEM((CHUNK, H), jnp.float32),
        pltpu.VMEM_SHARED((NP, H), jnp.float32),
    ],
)
def _deg_pass(dst_hbm, deg_hbm, didx, obuf, deg2d):
    c = lax.axis_index("c")
    s = lax.axis_index("s")
    wid = c * NS + s

    _zero_rows(obuf, IC, H // 16)
    for j in range(RPT // IC):
        pltpu.sync_copy(obuf.at[pl.ds(0, IC)],
                        deg2d.at[pl.ds(s * RPT + j * IC, IC)])
    _fill_ones(obuf, CHUNK, H // 16)
    plsc.subcore_barrier()

    base0 = wid * EPW

    def chunk(i, _):
        eb = pl.multiple_of(base0 + i * CHUNK, CHUNK)
        pltpu.sync_copy(dst_hbm.at[pl.ds(eb, CHUNK)], didx)
        pltpu.sync_copy(obuf, deg2d.at[didx], add=True)
        return 0

    lax.fori_loop(0, CH_PER_W, chunk, 0)
    plsc.subcore_barrier()

    rb = s * RPT
    pltpu.sync_copy(deg2d.at[pl.ds(rb, RPT)], deg_hbm.at[c, pl.ds(rb, RPT)])


@functools.partial(
    pl.kernel,
    out_type=jax.ShapeDtypeStruct((NC, NP, H), jnp.float32),
    mesh=_mesh,
    scratch_types=[
        pltpu.VMEM((SLAB, CHUNK), jnp.int32),
        pltpu.VMEM((SLAB, CHUNK), jnp.int32),
        pltpu.VMEM((CHUNK, H), jnp.float32),
        pltpu.VMEM((CHUNK, H), jnp.float32),
        pltpu.VMEM_SHARED((NP, H), jnp.float32),
        pltpu.SemaphoreType.DMA,
        pltpu.SemaphoreType.DMA,
    ],
)
def _conv_pass(src_hbm, dst_hbm, hs_hbm, out_hbm, sidx2, didx2, rows_a, rows_b,
               acc, sem0, sem1):
    c = lax.axis_index("c")
    s = lax.axis_index("s")
    wid = c * NS + s

    pltpu.async_copy(src_hbm.at[wid, pl.ds(0, SLAB)], sidx2, sem0)
    pltpu.async_copy(dst_hbm.at[wid, pl.ds(0, SLAB)], didx2, sem1)

    _zero_rows(rows_a, CHUNK, H // 16)
    for j in range(RPT // IC):
        pltpu.sync_copy(rows_a,
                        acc.at[pl.ds(s * RPT + j * IC, IC)])
    pltpu.make_async_copy(src_hbm.at[wid, pl.ds(0, SLAB)], sidx2, sem0).wait()
    pltpu.make_async_copy(dst_hbm.at[wid, pl.ds(0, SLAB)], didx2, sem1).wait()
    plsc.subcore_barrier()

    sems = (sem0, sem1)
    bufs = (rows_a, rows_b)

    for half in range(CH_PER_W // SLAB):
        if half > 0:
            hb = pl.multiple_of(half * SLAB, 8)
            pltpu.sync_copy(src_hbm.at[wid, pl.ds(hb, SLAB)], sidx2)
            pltpu.sync_copy(dst_hbm.at[wid, pl.ds(hb, SLAB)], didx2)
        pltpu.async_copy(hs_hbm.at[sidx2.at[0]], rows_a, sem0)

        def outer(j, _):
            for b in range(2):
                ch = 2 * j + b
                @pl.when(ch < SLAB - 1)
                def _():
                    pltpu.async_copy(hs_hbm.at[sidx2.at[ch + 1]],
                                     bufs[1 - b], sems[1 - b])
                pltpu.make_async_copy(hs_hbm.at[sidx2.at[ch]],
                                      bufs[b], sems[b]).wait()
                pltpu.sync_copy(bufs[b], acc.at[didx2.at[ch]], add=True)
            return 0

        lax.fori_loop(0, SLAB // 2, outer, 0)

    plsc.subcore_barrier()

    rb = s * RPT
    pltpu.sync_copy(acc.at[pl.ds(rb, RPT)], out_hbm.at[c, pl.ds(rb, RPT)])


@functools.partial(
    pl.kernel,
    out_type=(jax.ShapeDtypeStruct((G, H), jnp.float32),
              jax.ShapeDtypeStruct((G, H), jnp.float32)),
    mesh=_mesh,
    scratch_types=[
        pltpu.VMEM((PC,), jnp.int32),
        pltpu.VMEM((PC, H), jnp.float32),
        pltpu.VMEM((PC, H), jnp.float32),
        pltpu.VMEM_SHARED((G, H), jnp.float32),
        pltpu.VMEM_SHARED((G, H), jnp.float32),
    ],
)
def _pool_pass(h_hbm, batch_hbm, pooled_hbm, counts_hbm,
               bidx, prow, cbuf, pooled2d, counts2d):
    c = lax.axis_index("c")
    s = lax.axis_index("s")
    nch = (PNCH - s + NS - 1) // NS

    @pl.when(c == 0)
    def _():
        _zero_rows(prow, GPT, H // 16)
        pltpu.sync_copy(prow.at[pl.ds(0, GPT)],
                        pooled2d.at[pl.ds(s * GPT, GPT)])
        plsc.subcore_barrier()

        def chunk(j, _):
            eb = pl.multiple_of((s + j * NS) * PC, PC)
            pltpu.sync_copy(batch_hbm.at[pl.ds(eb, PC)], bidx)
            pltpu.sync_copy(h_hbm.at[pl.ds(eb, PC)], prow)
            pltpu.sync_copy(prow, pooled2d.at[bidx], add=True)
            return 0

        lax.fori_loop(0, nch, chunk, 0)
        plsc.subcore_barrier()
        pltpu.sync_copy(pooled2d.at[pl.ds(s * GPT, GPT)],
                        pooled_hbm.at[pl.ds(s * GPT, GPT)])

    @pl.when(c == 1)
    def _():
        _zero_rows(cbuf, GPT, H // 16)
        pltpu.sync_copy(cbuf.at[pl.ds(0, GPT)],
                        counts2d.at[pl.ds(s * GPT, GPT)])
        _fill_ones(cbuf, PC, H // 16)
        plsc.subcore_barrier()

        def chunk(j, _):
            eb = pl.multiple_of((s + j * NS) * PC, PC)
            pltpu.sync_copy(batch_hbm.at[pl.ds(eb, PC)], bidx)
            pltpu.sync_copy(cbuf, counts2d.at[bidx], add=True)
            return 0

        lax.fori_loop(0, nch, chunk, 0)
        plsc.subcore_barrier()
        pltpu.sync_copy(counts2d.at[pl.ds(s * GPT, GPT)],
                        counts_hbm.at[pl.ds(s * GPT, GPT)])


BLK = 2000


def _dinv_blk(d0_ref, d1_ref):
    deg = d0_ref[0, :, 0:1] + d1_ref[0, :, 0:1] + 1.0
    return lax.rsqrt(deg)


def _embed_body(x_ref, tab_ref, w_ref, out_ref):
    h = jnp.zeros((BLK, H), jnp.float32)
    for f in range(NF):
        xf = x_ref[:, f:f + 1]
        oh = (xf == lax.broadcasted_iota(jnp.int32, (BLK, V), 1))
        h = h + jnp.dot(oh.astype(jnp.float32), tab_ref[f],
                        preferred_element_type=jnp.float32)
    out_ref[...] = jnp.dot(h, w_ref[...],
                           preferred_element_type=jnp.float32)


def _scale_body(hw_ref, d0_ref, d1_ref, out_ref):
    out_ref[...] = hw_ref[...] * _dinv_blk(d0_ref, d1_ref)


def _tcl_body(p0_ref, p1_ref, hs_ref, d0_ref, d1_ref, b_ref, w_ref, out_ref,
              *, relu):
    dinv = _dinv_blk(d0_ref, d1_ref)
    t = (p0_ref[0] + p1_ref[0] + hs_ref[...]) * dinv + b_ref[...]
    if relu:
        t = jnp.maximum(t, 0.0)
    out_ref[...] = jnp.dot(t, w_ref[...],
                           preferred_element_type=jnp.float32) * dinv


def _tc3_body(p0_ref, p1_ref, hs_ref, d0_ref, d1_ref, b_ref, out_ref):
    dinv = _dinv_blk(d0_ref, d1_ref)
    out_ref[...] = (p0_ref[0] + p1_ref[0] + hs_ref[...]) * dinv + b_ref[...]


def _tc4_body(pp_ref, cc_ref, lw_ref, lb_ref, out_ref):
    cnt = jnp.maximum(cc_ref[:, 0:1], 1.0)
    pooled = pp_ref[...] / cnt
    logit = jnp.sum(pooled * lw_ref[...], axis=1, keepdims=True) + lb_ref[0, 0]
    out_ref[...] = jax.nn.sigmoid(logit)


_row_spec = pl.BlockSpec((BLK, H), lambda i: (i, 0))
_part0_spec = pl.BlockSpec((1, BLK, H), lambda i: (0, i, 0))
_part1_spec = pl.BlockSpec((1, BLK, H), lambda i: (1, i, 0))
_deg0_spec = pl.BlockSpec((1, BLK, H), lambda i: (0, i, 0))
_deg1_spec = pl.BlockSpec((1, BLK, H), lambda i: (1, i, 0))
_w_spec = pl.BlockSpec((H, H), lambda i: (0, 0))
_b_spec = pl.BlockSpec((1, H), lambda i: (0, 0))
_node_out = jax.ShapeDtypeStruct((N, H), jnp.float32)


def _tc_embed(x, tables, W1):
    return pl.pallas_call(
        _embed_body,
        grid=(N // BLK,),
        in_specs=[
            pl.BlockSpec((BLK, NF), lambda i: (i, 0)),
            pl.BlockSpec((NF, V, H), lambda i: (0, 0, 0)),
            _w_spec,
        ],
        out_specs=_row_spec,
        out_shape=_node_out,
    )(x, tables, W1)


def _tc_scale(hw, deg):
    return pl.pallas_call(
        _scale_body,
        grid=(N // BLK,),
        in_specs=[_row_spec, _deg0_spec, _deg1_spec],
        out_specs=_row_spec,
        out_shape=_node_out,
    )(hw, deg, deg)


def _tcl(parts, hs, deg, b, Wn, relu):
    return pl.pallas_call(
        functools.partial(_tcl_body, relu=relu),
        grid=(N // BLK,),
        in_specs=[_part0_spec, _part1_spec, _row_spec, _deg0_spec, _deg1_spec,
                  _b_spec, _w_spec],
        out_specs=_row_spec,
        out_shape=_node_out,
    )(parts, parts, hs, deg, deg, b.reshape(1, H), Wn)


def _tc3(parts, hs, deg, b):
    return pl.pallas_call(
        _tc3_body,
        grid=(N // BLK,),
        in_specs=[_part0_spec, _part1_spec, _row_spec, _deg0_spec, _deg1_spec,
                  _b_spec],
        out_specs=_row_spec,
        out_shape=_node_out,
    )(parts, parts, hs, deg, deg, b.reshape(1, H))


def _tc4(pooled, counts, lin_w, lin_b):
    return pl.pallas_call(
        _tc4_body,
        in_specs=[
            pl.BlockSpec((G, H), lambda: (0, 0)),
            pl.BlockSpec((G, H), lambda: (0, 0)),
            pl.BlockSpec((1, H), lambda: (0, 0)),
            pl.BlockSpec((1, 1), lambda: (0, 0)),
        ],
        out_specs=pl.BlockSpec((G, 1), lambda: (0, 0)),
        out_shape=jax.ShapeDtypeStruct((G, 1), jnp.float32),
    )(pooled, counts, lin_w.reshape(1, H), lin_b.reshape(1, 1))


def kernel(x, edge_index, batch, tables, W1, b1, W2, b2, W3, b3, lin_w, lin_b):
    src = edge_index[0].astype(jnp.int32)
    dst = edge_index[1].astype(jnp.int32)
    npad = E_PAD - E
    src_pad = jnp.concatenate(
        [src, jnp.arange(npad, dtype=jnp.int32) * 37 % N])
    dst_pad = jnp.concatenate(
        [dst, N + jnp.arange(npad, dtype=jnp.int32) % (NP - N)])
    src3 = src_pad.reshape(NW, CH_PER_W, CHUNK)
    dst3 = dst_pad.reshape(NW, CH_PER_W, CHUNK)

    deg = _deg_pass(dst_pad)
    hw1 = _tc_embed(x.astype(jnp.int32), tables, W1)

    hs1 = _tc_scale(hw1, deg)
    parts = _conv_pass(src3, dst3, hs1)
    hs2 = _tcl(parts, hs1, deg, b1, W2, relu=True)
    parts = _conv_pass(src3, dst3, hs2)
    hs3 = _tcl(parts, hs2, deg, b2, W3, relu=True)
    parts = _conv_pass(src3, dst3, hs3)
    h_out = _tc3(parts, hs3, deg, b3)

    pooled, counts = _pool_pass(h_out, batch.astype(jnp.int32))
    return _tc4(pooled, counts, lin_w, lin_b)
